# baseline (speedup 1.0000x reference)
import functools

import jax
import jax.numpy as jnp
from jax import lax
from jax.experimental import pallas as pl
from jax.experimental.pallas import tpu as pltpu

_B, _C, _H, _W = 4, 19, 512, 512
_N = _B * _H * _W
_K = 100000
_THRESH = 0.7
_HB = 64
_GRID = (_B, _H // _HB)


def _softmax_ce(preds_ref, targets_ref, b):
    m = preds_ref[b, 0]
    tmax = targets_ref[b, 0]
    psel = m
    for c in range(1, _C):
        pc = preds_ref[b, c]
        m = jnp.maximum(m, pc)
        tc = targets_ref[b, c]
        upd = tc > tmax
        psel = jnp.where(upd, pc, psel)
        tmax = jnp.where(upd, tc, tmax)
    s = jnp.zeros_like(m)
    for c in range(_C):
        s = s + jnp.exp(preds_ref[b, c] - m)
    inv_s = 1.0 / s
    pred_t = jnp.exp(psel - m) * inv_s
    z = jnp.zeros_like(m)
    for c in range(_C):
        z = z + jnp.exp(jnp.exp(preds_ref[b, c] - m) * inv_s)
    loss = jnp.log(z) - pred_t
    return pred_t, loss


def _fast_body(preds_ref, targets_ref, out_ref, sum_acc, cnt_acc):
    step = pl.program_id(0) * pl.num_programs(1) + pl.program_id(1)

    @pl.when(step == 0)
    def _():
        sum_acc[...] = jnp.zeros_like(sum_acc)
        cnt_acc[...] = jnp.zeros_like(cnt_acc)

    pred_t, loss = _softmax_ce(preds_ref, targets_ref, 0)
    keep = pred_t < jnp.float32(_THRESH)
    w = jnp.where(keep, loss, 0.0)
    cnt = keep.astype(jnp.float32)
    ws = w[0:8]
    cs = cnt[0:8]
    for r in range(8, _HB, 8):
        ws = ws + w[r:r + 8]
        cs = cs + cnt[r:r + 8]
    sum_acc[...] += ws
    cnt_acc[...] += cs

    @pl.when(step == _GRID[0] * _GRID[1] - 1)
    def _():
        out_ref[0, 0] = jnp.sum(sum_acc[...])
        out_ref[0, 1] = jnp.sum(cnt_acc[...])


def _fast_stats(preds, targets):
    in_spec = pl.BlockSpec((1, _C, _HB, _W), lambda b, h: (b, 0, h, 0))
    return pl.pallas_call(
        _fast_body,
        grid=_GRID,
        in_specs=[in_spec, in_spec],
        out_specs=pl.BlockSpec(memory_space=pltpu.SMEM),
        out_shape=jax.ShapeDtypeStruct((1, 2), jnp.float32),
        scratch_shapes=[
            pltpu.VMEM((8, _W), jnp.float32),
            pltpu.VMEM((8, _W), jnp.float32),
        ],
        compiler_params=pltpu.CompilerParams(
            dimension_semantics=("arbitrary", "arbitrary"),
        ),
    )(preds, targets)


def _ce_out_body(preds_ref, targets_ref, predt_ref, loss_ref):
    pred_t, loss = _softmax_ce(preds_ref, targets_ref, 0)
    predt_ref[0] = pred_t
    loss_ref[0] = loss


_ROWS, _COLS = 1024, 1024
_CH = 32
_NCHUNK = _ROWS // _CH


def _select_body(predt_ref, loss_ref, out_ref):
    def count_le(mid):
        def chunk(i, acc):
            blk = predt_ref[pl.ds(i * _CH, _CH), :]
            bits = lax.bitcast_convert_type(blk, jnp.int32)
            mask = (bits <= mid).astype(jnp.int32)
            part = mask[0:8] + mask[8:16] + mask[16:24] + mask[24:32]
            return acc + part
        acc = lax.fori_loop(0, _NCHUNK, chunk,
                            jnp.zeros((8, _COLS), jnp.int32), unroll=2)
        return jnp.sum(acc)

    def bstep(_, carry):
        lo, hi = carry
        mid = lax.div(lo + hi, jnp.int32(2))
        pred = count_le(mid) >= jnp.int32(_K + 1)
        return jnp.where(pred, lo, mid), jnp.where(pred, mid, hi)

    lo0 = jnp.int32(-1)
    hi0 = jnp.int32(0x3F800000)
    _, hi = lax.fori_loop(0, 30, bstep, (lo0, hi0))
    kth = lax.bitcast_convert_type(hi, jnp.float32)
    thr = jnp.maximum(kth, jnp.float32(_THRESH))

    def acc_chunk(i, carry):
        ksum, kcnt = carry
        pt = predt_ref[pl.ds(i * _CH, _CH), :]
        ls = loss_ref[pl.ds(i * _CH, _CH), :]
        keep = pt < thr
        ls = jnp.where(keep, ls, 0.0)
        cnt = keep.astype(jnp.float32)
        ksum = ksum + (ls[0:8] + ls[8:16] + ls[16:24] + ls[24:32])
        kcnt = kcnt + (cnt[0:8] + cnt[8:16] + cnt[16:24] + cnt[24:32])
        return ksum, kcnt

    z8 = jnp.zeros((8, _COLS), jnp.float32)
    ksum, kcnt = lax.fori_loop(0, _NCHUNK, acc_chunk, (z8, z8), unroll=2)
    out_ref[0, 0] = jnp.sum(ksum) / jnp.sum(kcnt)


def _slow_path(preds, targets):
    in_spec = pl.BlockSpec((1, _C, _HB, _W), lambda b, h: (b, 0, h, 0))
    out_spec = pl.BlockSpec((1, _HB, _W), lambda b, h: (b, h, 0))
    pred_t, loss = pl.pallas_call(
        _ce_out_body,
        grid=_GRID,
        in_specs=[in_spec, in_spec],
        out_specs=[out_spec, out_spec],
        out_shape=[
            jax.ShapeDtypeStruct((_B, _H, _W), jnp.float32),
            jax.ShapeDtypeStruct((_B, _H, _W), jnp.float32),
        ],
        compiler_params=pltpu.CompilerParams(
            dimension_semantics=("parallel", "parallel"),
        ),
    )(preds, targets)
    out = pl.pallas_call(
        _select_body,
        in_specs=[
            pl.BlockSpec((_ROWS, _COLS), lambda: (0, 0)),
            pl.BlockSpec((_ROWS, _COLS), lambda: (0, 0)),
        ],
        out_specs=pl.BlockSpec(memory_space=pltpu.SMEM),
        out_shape=jax.ShapeDtypeStruct((1, 1), jnp.float32),
    )(pred_t.reshape(_ROWS, _COLS), loss.reshape(_ROWS, _COLS))
    return out[0, 0]


@jax.jit
def kernel(preds, targets):
    stats = _fast_stats(preds, targets)
    ksum, kcnt = stats[0, 0], stats[0, 1]
    return lax.cond(
        kcnt >= jnp.float32(_K + 1),
        lambda: ksum / kcnt,
        lambda: _slow_path(preds, targets),
    )


# speedup vs baseline: 1.0845x; 1.0845x over previous
import functools

import jax
import jax.numpy as jnp
from jax import lax
from jax.experimental import pallas as pl
from jax.experimental.pallas import tpu as pltpu

_B, _C, _H, _W = 4, 19, 512, 512
_N = _B * _H * _W
_K = 100000
_THRESH = 0.7
_HB = 128
_GRID = (_B, _H // _HB)


def _softmax_ce(preds_ref, targets_ref, b):
    m = preds_ref[b, 0]
    tmax = targets_ref[b, 0]
    psel = m
    for c in range(1, _C):
        pc = preds_ref[b, c]
        m = jnp.maximum(m, pc)
        tc = targets_ref[b, c]
        upd = tc > tmax
        psel = jnp.where(upd, pc, psel)
        tmax = jnp.where(upd, tc, tmax)
    s = jnp.zeros_like(m)
    for c in range(_C):
        s = s + jnp.exp(preds_ref[b, c] - m)
    inv_s = 1.0 / s
    pred_t = jnp.exp(psel - m) * inv_s
    z = jnp.zeros_like(m)
    for c in range(_C):
        z = z + jnp.exp(jnp.exp(preds_ref[b, c] - m) * inv_s)
    loss = jnp.log(z) - pred_t
    return pred_t, loss


def _fast_body(preds_ref, targets_ref, out_ref, sum_acc, cnt_acc):
    step = pl.program_id(0) * pl.num_programs(1) + pl.program_id(1)

    @pl.when(step == 0)
    def _():
        sum_acc[...] = jnp.zeros_like(sum_acc)
        cnt_acc[...] = jnp.zeros_like(cnt_acc)

    pred_t, loss = _softmax_ce(preds_ref, targets_ref, 0)
    keep = pred_t < jnp.float32(_THRESH)
    w = jnp.where(keep, loss, 0.0)
    cnt = keep.astype(jnp.float32)
    ws = w[0:8]
    cs = cnt[0:8]
    for r in range(8, _HB, 8):
        ws = ws + w[r:r + 8]
        cs = cs + cnt[r:r + 8]
    sum_acc[...] += ws
    cnt_acc[...] += cs

    @pl.when(step == _GRID[0] * _GRID[1] - 1)
    def _():
        out_ref[0, 0] = jnp.sum(sum_acc[...])
        out_ref[0, 1] = jnp.sum(cnt_acc[...])


def _fast_stats(preds, targets):
    in_spec = pl.BlockSpec((1, _C, _HB, _W), lambda b, h: (b, 0, h, 0))
    return pl.pallas_call(
        _fast_body,
        grid=_GRID,
        in_specs=[in_spec, in_spec],
        out_specs=pl.BlockSpec(memory_space=pltpu.SMEM),
        out_shape=jax.ShapeDtypeStruct((1, 2), jnp.float32),
        scratch_shapes=[
            pltpu.VMEM((8, _W), jnp.float32),
            pltpu.VMEM((8, _W), jnp.float32),
        ],
        compiler_params=pltpu.CompilerParams(
            dimension_semantics=("arbitrary", "arbitrary"),
        ),
    )(preds, targets)


def _ce_out_body(preds_ref, targets_ref, predt_ref, loss_ref):
    pred_t, loss = _softmax_ce(preds_ref, targets_ref, 0)
    predt_ref[0] = pred_t
    loss_ref[0] = loss


_ROWS, _COLS = 1024, 1024
_CH = 32
_NCHUNK = _ROWS // _CH


def _select_body(predt_ref, loss_ref, out_ref):
    def count_le(mid):
        def chunk(i, acc):
            blk = predt_ref[pl.ds(i * _CH, _CH), :]
            bits = lax.bitcast_convert_type(blk, jnp.int32)
            mask = (bits <= mid).astype(jnp.int32)
            part = mask[0:8] + mask[8:16] + mask[16:24] + mask[24:32]
            return acc + part
        acc = lax.fori_loop(0, _NCHUNK, chunk,
                            jnp.zeros((8, _COLS), jnp.int32), unroll=2)
        return jnp.sum(acc)

    def bstep(_, carry):
        lo, hi = carry
        mid = lax.div(lo + hi, jnp.int32(2))
        pred = count_le(mid) >= jnp.int32(_K + 1)
        return jnp.where(pred, lo, mid), jnp.where(pred, mid, hi)

    lo0 = jnp.int32(-1)
    hi0 = jnp.int32(0x3F800000)
    _, hi = lax.fori_loop(0, 30, bstep, (lo0, hi0))
    kth = lax.bitcast_convert_type(hi, jnp.float32)
    thr = jnp.maximum(kth, jnp.float32(_THRESH))

    def acc_chunk(i, carry):
        ksum, kcnt = carry
        pt = predt_ref[pl.ds(i * _CH, _CH), :]
        ls = loss_ref[pl.ds(i * _CH, _CH), :]
        keep = pt < thr
        ls = jnp.where(keep, ls, 0.0)
        cnt = keep.astype(jnp.float32)
        ksum = ksum + (ls[0:8] + ls[8:16] + ls[16:24] + ls[24:32])
        kcnt = kcnt + (cnt[0:8] + cnt[8:16] + cnt[16:24] + cnt[24:32])
        return ksum, kcnt

    z8 = jnp.zeros((8, _COLS), jnp.float32)
    ksum, kcnt = lax.fori_loop(0, _NCHUNK, acc_chunk, (z8, z8), unroll=2)
    out_ref[0, 0] = jnp.sum(ksum) / jnp.sum(kcnt)


def _slow_path(preds, targets):
    in_spec = pl.BlockSpec((1, _C, _HB, _W), lambda b, h: (b, 0, h, 0))
    out_spec = pl.BlockSpec((1, _HB, _W), lambda b, h: (b, h, 0))
    pred_t, loss = pl.pallas_call(
        _ce_out_body,
        grid=_GRID,
        in_specs=[in_spec, in_spec],
        out_specs=[out_spec, out_spec],
        out_shape=[
            jax.ShapeDtypeStruct((_B, _H, _W), jnp.float32),
            jax.ShapeDtypeStruct((_B, _H, _W), jnp.float32),
        ],
        compiler_params=pltpu.CompilerParams(
            dimension_semantics=("parallel", "parallel"),
        ),
    )(preds, targets)
    out = pl.pallas_call(
        _select_body,
        in_specs=[
            pl.BlockSpec((_ROWS, _COLS), lambda: (0, 0)),
            pl.BlockSpec((_ROWS, _COLS), lambda: (0, 0)),
        ],
        out_specs=pl.BlockSpec(memory_space=pltpu.SMEM),
        out_shape=jax.ShapeDtypeStruct((1, 1), jnp.float32),
    )(pred_t.reshape(_ROWS, _COLS), loss.reshape(_ROWS, _COLS))
    return out[0, 0]


@jax.jit
def kernel(preds, targets):
    stats = _fast_stats(preds, targets)
    ksum, kcnt = stats[0, 0], stats[0, 1]
    return lax.cond(
        kcnt >= jnp.float32(_K + 1),
        lambda: ksum / kcnt,
        lambda: _slow_path(preds, targets),
    )


# speedup vs baseline: 1.2271x; 1.1315x over previous
import functools

import jax
import jax.numpy as jnp
from jax import lax
from jax.experimental import pallas as pl
from jax.experimental.pallas import tpu as pltpu

_B, _C, _H, _W = 4, 19, 512, 512
_N = _B * _H * _W
_K = 100000
_THRESH = 0.7
_HB = 128
_GRID = (_B, _H // _HB)


_RT = 8


def _softmax_ce_tile(preds_ref, targets_ref, r0):
    rs = pl.ds(r0, _RT)
    m = preds_ref[0, 0, rs]
    tmax = targets_ref[0, 0, rs]
    psel = m
    for c in range(1, _C):
        pc = preds_ref[0, c, rs]
        m = jnp.maximum(m, pc)
        tc = targets_ref[0, c, rs]
        upd = tc > tmax
        psel = jnp.where(upd, pc, psel)
        tmax = jnp.where(upd, tc, tmax)
    s = jnp.zeros_like(m)
    for c in range(_C):
        s = s + jnp.exp(preds_ref[0, c, rs] - m)
    inv_s = 1.0 / s
    pred_t = jnp.exp(psel - m) * inv_s
    z = jnp.zeros_like(m)
    for c in range(_C):
        z = z + jnp.exp(jnp.exp(preds_ref[0, c, rs] - m) * inv_s)
    loss = jnp.log(z) - pred_t
    return pred_t, loss


def _fast_body(preds_ref, targets_ref, out_ref, sum_acc, cnt_acc):
    step = pl.program_id(0) * pl.num_programs(1) + pl.program_id(1)

    @pl.when(step == 0)
    def _():
        sum_acc[...] = jnp.zeros_like(sum_acc)
        cnt_acc[...] = jnp.zeros_like(cnt_acc)

    ws = jnp.zeros((_RT, _W), jnp.float32)
    cs = jnp.zeros((_RT, _W), jnp.float32)
    for r0 in range(0, _HB, _RT):
        pred_t, loss = _softmax_ce_tile(preds_ref, targets_ref, r0)
        keep = pred_t < jnp.float32(_THRESH)
        ws = ws + jnp.where(keep, loss, 0.0)
        cs = cs + keep.astype(jnp.float32)
    sum_acc[...] += ws
    cnt_acc[...] += cs

    @pl.when(step == _GRID[0] * _GRID[1] - 1)
    def _():
        out_ref[0, 0] = jnp.sum(sum_acc[...])
        out_ref[0, 1] = jnp.sum(cnt_acc[...])


def _fast_stats(preds, targets):
    in_spec = pl.BlockSpec((1, _C, _HB, _W), lambda b, h: (b, 0, h, 0))
    return pl.pallas_call(
        _fast_body,
        grid=_GRID,
        in_specs=[in_spec, in_spec],
        out_specs=pl.BlockSpec(memory_space=pltpu.SMEM),
        out_shape=jax.ShapeDtypeStruct((1, 2), jnp.float32),
        scratch_shapes=[
            pltpu.VMEM((8, _W), jnp.float32),
            pltpu.VMEM((8, _W), jnp.float32),
        ],
        compiler_params=pltpu.CompilerParams(
            dimension_semantics=("arbitrary", "arbitrary"),
        ),
    )(preds, targets)


def _ce_out_body(preds_ref, targets_ref, predt_ref, loss_ref):
    for r0 in range(0, _HB, _RT):
        pred_t, loss = _softmax_ce_tile(preds_ref, targets_ref, r0)
        predt_ref[0, pl.ds(r0, _RT)] = pred_t
        loss_ref[0, pl.ds(r0, _RT)] = loss


_ROWS, _COLS = 1024, 1024
_CH = 32
_NCHUNK = _ROWS // _CH


def _select_body(predt_ref, loss_ref, out_ref):
    def count_le(mid):
        def chunk(i, acc):
            blk = predt_ref[pl.ds(i * _CH, _CH), :]
            bits = lax.bitcast_convert_type(blk, jnp.int32)
            mask = (bits <= mid).astype(jnp.int32)
            part = mask[0:8] + mask[8:16] + mask[16:24] + mask[24:32]
            return acc + part
        acc = lax.fori_loop(0, _NCHUNK, chunk,
                            jnp.zeros((8, _COLS), jnp.int32), unroll=2)
        return jnp.sum(acc)

    def bstep(_, carry):
        lo, hi = carry
        mid = lax.div(lo + hi, jnp.int32(2))
        pred = count_le(mid) >= jnp.int32(_K + 1)
        return jnp.where(pred, lo, mid), jnp.where(pred, mid, hi)

    lo0 = jnp.int32(-1)
    hi0 = jnp.int32(0x3F800000)
    _, hi = lax.fori_loop(0, 30, bstep, (lo0, hi0))
    kth = lax.bitcast_convert_type(hi, jnp.float32)
    thr = jnp.maximum(kth, jnp.float32(_THRESH))

    def acc_chunk(i, carry):
        ksum, kcnt = carry
        pt = predt_ref[pl.ds(i * _CH, _CH), :]
        ls = loss_ref[pl.ds(i * _CH, _CH), :]
        keep = pt < thr
        ls = jnp.where(keep, ls, 0.0)
        cnt = keep.astype(jnp.float32)
        ksum = ksum + (ls[0:8] + ls[8:16] + ls[16:24] + ls[24:32])
        kcnt = kcnt + (cnt[0:8] + cnt[8:16] + cnt[16:24] + cnt[24:32])
        return ksum, kcnt

    z8 = jnp.zeros((8, _COLS), jnp.float32)
    ksum, kcnt = lax.fori_loop(0, _NCHUNK, acc_chunk, (z8, z8), unroll=2)
    out_ref[0, 0] = jnp.sum(ksum) / jnp.sum(kcnt)


def _slow_path(preds, targets):
    in_spec = pl.BlockSpec((1, _C, _HB, _W), lambda b, h: (b, 0, h, 0))
    out_spec = pl.BlockSpec((1, _HB, _W), lambda b, h: (b, h, 0))
    pred_t, loss = pl.pallas_call(
        _ce_out_body,
        grid=_GRID,
        in_specs=[in_spec, in_spec],
        out_specs=[out_spec, out_spec],
        out_shape=[
            jax.ShapeDtypeStruct((_B, _H, _W), jnp.float32),
            jax.ShapeDtypeStruct((_B, _H, _W), jnp.float32),
        ],
        compiler_params=pltpu.CompilerParams(
            dimension_semantics=("parallel", "parallel"),
        ),
    )(preds, targets)
    out = pl.pallas_call(
        _select_body,
        in_specs=[
            pl.BlockSpec((_ROWS, _COLS), lambda: (0, 0)),
            pl.BlockSpec((_ROWS, _COLS), lambda: (0, 0)),
        ],
        out_specs=pl.BlockSpec(memory_space=pltpu.SMEM),
        out_shape=jax.ShapeDtypeStruct((1, 1), jnp.float32),
    )(pred_t.reshape(_ROWS, _COLS), loss.reshape(_ROWS, _COLS))
    return out[0, 0]


@jax.jit
def kernel(preds, targets):
    stats = _fast_stats(preds, targets)
    ksum, kcnt = stats[0, 0], stats[0, 1]
    return lax.cond(
        kcnt >= jnp.float32(_K + 1),
        lambda: ksum / kcnt,
        lambda: _slow_path(preds, targets),
    )
